# Initial kernel scaffold; baseline (speedup 1.0000x reference)
import jax
import jax.numpy as jnp
from jax.experimental import pallas as pl


def kernel(x, pokemon_table, move_table, W1, b1, W2, b2, W3, b3):
    raise NotImplementedError("write your pallas kernel here")


# speedup vs baseline: 3.0806x; 3.0806x over previous
import jax
import jax.numpy as jnp
from jax import lax
from jax.experimental import pallas as pl


BB = 512


def _body(x_ref, pt_ref, mt_ref, w1_ref, b1_ref, w2_ref, b2_ref, w3_ref,
          b3_ref, out_ref):
    xb = x_ref[:, :]
    iota = lax.broadcasted_iota(jnp.int32, (BB, 1024), 1)

    def emb(tab_ref, col):
        idx = xb[:, col:col + 1].astype(jnp.int32)
        oh = (idx == iota).astype(jnp.float32)
        return jnp.dot(oh, tab_ref[:, :], preferred_element_type=jnp.float32)

    h1 = jnp.dot(emb(pt_ref, 0), w1_ref[0:32, :],
                 preferred_element_type=jnp.float32)
    h1 += jnp.dot(emb(pt_ref, 1), w1_ref[32:64, :],
                  preferred_element_type=jnp.float32)
    for k in range(4):
        h1 += jnp.dot(emb(mt_ref, 2 + k), w1_ref[64 + 32 * k:96 + 32 * k, :],
                      preferred_element_type=jnp.float32)
    h1 += jnp.dot(xb[:, 6:10], w1_ref[192:196, :],
                  preferred_element_type=jnp.float32)
    h1 = jnp.maximum(h1 + b1_ref[:, :], 0.0)
    h2 = jnp.maximum(jnp.dot(h1, w2_ref[:, :],
                             preferred_element_type=jnp.float32)
                     + b2_ref[:, :], 0.0)
    out_ref[:, :] = (jnp.dot(h2, w3_ref[:, :],
                             preferred_element_type=jnp.float32)
                     + b3_ref[:, :])


def kernel(x, pokemon_table, move_table, W1, b1, W2, b2, W3, b3):
    B = x.shape[0]
    pt = jnp.pad(pokemon_table, ((0, 1024 - pokemon_table.shape[0]), (0, 0)))
    mt = jnp.pad(move_table, ((0, 1024 - move_table.shape[0]), (0, 0)))
    grid = B // BB
    full = lambda shape: pl.BlockSpec(shape, lambda i: (0, 0))
    return pl.pallas_call(
        _body,
        grid=(grid,),
        in_specs=[
            pl.BlockSpec((BB, 10), lambda i: (i, 0)),
            full(pt.shape),
            full(mt.shape),
            full(W1.shape),
            full((1, 128)),
            full(W2.shape),
            full((1, 64)),
            full(W3.shape),
            full((1, 18)),
        ],
        out_specs=pl.BlockSpec((BB, 18), lambda i: (i, 0)),
        out_shape=jax.ShapeDtypeStruct((B, 18), jnp.float32),
    )(x, pt, mt, W1, b1.reshape(1, 128), W2, b2.reshape(1, 64), W3,
      b3.reshape(1, 18))


# speedup vs baseline: 4.1760x; 1.3556x over previous
import jax
import jax.numpy as jnp
from jax import lax
from jax.experimental import pallas as pl
from jax.experimental.pallas import tpu as pltpu
from jax.experimental.pallas import tpu_sc as plsc

NC = 2
NS = 16
NW = NC * NS
CH = 128
BB = 1024


def _sc_gather_body(pt_hbm, mt_hbm, idx_hbm, out_hbm, idx_v, rows_v, zero_v,
                    sem):
    wid = lax.axis_index("s") * NC + lax.axis_index("c")
    pltpu.sync_copy(idx_hbm.at[wid], idx_v)
    nch = idx_v.shape[0] // 6
    zero_v[...] = jnp.zeros_like(zero_v)
    copies = []
    for j in range(6):
        tab = pt_hbm if j < 2 else mt_hbm
        for k in range(nch):
            copies.append(
                pltpu.async_copy(tab.at[idx_v.at[j * nch + k]],
                                 rows_v.at[j, k], sem))
    for c in copies:
        c.wait()
    base = wid * (nch * CH)
    for k in range(nch):
        r0 = base + k * CH
        for j in range(6):
            pltpu.sync_copy(rows_v.at[j, k],
                            out_hbm.at[pl.ds(r0, CH), pl.ds(32 * j, 32)])
        pltpu.sync_copy(zero_v, out_hbm.at[pl.ds(r0, CH), pl.ds(192, 64)])


def _mlp_body(act_ref, x_ref, w1p_ref, w1n_ref, b1_ref, w2_ref, b2_ref,
              w3_ref, b3_ref, out_ref):
    h1 = jnp.dot(act_ref[:, :], w1p_ref[:, :],
                 preferred_element_type=jnp.float32)
    h1 += jnp.dot(x_ref[:, 6:10], w1n_ref[:, :],
                  preferred_element_type=jnp.float32)
    h1 = jnp.maximum(h1 + b1_ref[:, :], 0.0)
    h2 = jnp.maximum(jnp.dot(h1, w2_ref[:, :],
                             preferred_element_type=jnp.float32)
                     + b2_ref[:, :], 0.0)
    out_ref[:, :] = (jnp.dot(h2, w3_ref[:, :],
                             preferred_element_type=jnp.float32)
                     + b3_ref[:, :])


def kernel(x, pokemon_table, move_table, W1, b1, W2, b2, W3, b3):
    B = x.shape[0]
    bpw = B // NW
    nch = bpw // CH
    ids = x[:, :6].astype(jnp.int32)
    idx = ids.T.reshape(6, NW, nch, CH).transpose(1, 0, 2, 3)
    idx = idx.reshape(NW, 6 * nch, CH)
    pt = jnp.pad(pokemon_table, ((0, 1024 - pokemon_table.shape[0]), (0, 0)))
    mt = jnp.pad(move_table, ((0, 1024 - move_table.shape[0]), (0, 0)))

    acts = pl.kernel(
        _sc_gather_body,
        out_type=jax.ShapeDtypeStruct((B, 256), jnp.float32),
        scratch_types=[
            pltpu.VMEM((6 * nch, CH), jnp.int32),
            pltpu.VMEM((6, nch, CH, 32), jnp.float32),
            pltpu.VMEM((CH, 64), jnp.float32),
            pltpu.SemaphoreType.DMA,
        ],
        mesh=plsc.VectorSubcoreMesh(core_axis_name="c", subcore_axis_name="s"),
        compiler_params=pltpu.CompilerParams(use_tc_tiling_on_sc=False),
    )(pt, mt, idx)

    W1p = jnp.pad(W1[:192], ((0, 64), (0, 0)))
    W1n = W1[192:196]

    grid = B // BB
    full = lambda shape: pl.BlockSpec(shape, lambda i: (0,) * len(shape))
    return pl.pallas_call(
        _mlp_body,
        grid=(grid,),
        in_specs=[
            pl.BlockSpec((BB, 256), lambda i: (i, 0)),
            pl.BlockSpec((BB, 10), lambda i: (i, 0)),
            full((256, 128)),
            full((4, 128)),
            full((1, 128)),
            full(W2.shape),
            full((1, 64)),
            full(W3.shape),
            full((1, 18)),
        ],
        out_specs=pl.BlockSpec((BB, 18), lambda i: (i, 0)),
        out_shape=jax.ShapeDtypeStruct((B, 18), jnp.float32),
    )(acts, x, W1p, W1n, b1.reshape(1, 128), W2, b2.reshape(1, 64), W3,
      b3.reshape(1, 18))
